# baseline (speedup 1.0000x reference)
import jax
import jax.numpy as jnp
from jax import lax
from jax.experimental import pallas as pl
from jax.experimental.pallas import tpu as pltpu
from jax.experimental.pallas import tpu_sc as plsc

_B = 16384
_D = 20
_NC = 2
_NS = 16
_NW = _NC * _NS
_BPW = _B // _NW
_CHUNK = 128
_NCHUNK = _BPW // _CHUNK
_GROUPS = _BPW // 16
_EPS2 = 1e-16


def _rsqrt(p):
    i = plsc.bitcast(p, jnp.int32)
    i = jnp.int32(0x5F3759DF) - (i >> 1)
    y = plsc.bitcast(i, jnp.float32)
    for _ in range(3):
        y = y * (1.5 - 0.5 * p * y * y)
    return y


def _body(users, movies, uf, mf, out, idx_u, idx_m, u_rows, m_rows, out_v, sem):
    wid = lax.axis_index("s") * _NC + lax.axis_index("c")
    base = wid * _BPW
    pltpu.sync_copy(users.at[pl.ds(base, _BPW)], idx_u)
    pltpu.sync_copy(movies.at[pl.ds(base, _BPW)], idx_m)
    copies = []
    for j in range(_NCHUNK):
        sl = pl.ds(j * _CHUNK, _CHUNK)
        copies.append(pltpu.async_copy(uf.at[idx_u.at[sl]], u_rows.at[sl], sem))
        copies.append(pltpu.async_copy(mf.at[idx_m.at[sl]], m_rows.at[sl], sem))
    for c in copies:
        c.wait()

    lanes = lax.iota(jnp.int32, 16)

    def group(g, carry):
        rows = g * 16 + lanes
        dot = jnp.zeros((16,), jnp.float32)
        nu = jnp.zeros((16,), jnp.float32)
        nm = jnp.zeros((16,), jnp.float32)
        for d in range(_D):
            col = jnp.full((16,), d, jnp.int32)
            uv = plsc.load_gather(u_rows, [rows, col])
            mv = plsc.load_gather(m_rows, [rows, col])
            dot = dot + uv * mv
            nu = nu + uv * uv
            nm = nm + mv * mv
        p = jnp.maximum(nu, _EPS2) * jnp.maximum(nm, _EPS2)
        out_v[pl.ds(g * 16, 16)] = dot * _rsqrt(p) * 2.0 + 3.0
        return carry

    lax.fori_loop(0, _GROUPS, group, 0)
    pltpu.sync_copy(out_v, out.at[pl.ds(base, _BPW)])


def kernel(users, movies, user_factors, movie_factors):
    mesh = plsc.VectorSubcoreMesh(
        core_axis_name="c", subcore_axis_name="s",
        num_cores=_NC, num_subcores=_NS)
    f = pl.kernel(
        _body,
        out_type=jax.ShapeDtypeStruct((_B,), jnp.float32),
        mesh=mesh,
        scratch_types=[
            pltpu.VMEM((_BPW,), jnp.int32),
            pltpu.VMEM((_BPW,), jnp.int32),
            pltpu.VMEM((_BPW, _D), jnp.float32),
            pltpu.VMEM((_BPW, _D), jnp.float32),
            pltpu.VMEM((_BPW,), jnp.float32),
            pltpu.SemaphoreType.DMA,
        ],
        compiler_params=pltpu.CompilerParams(
            needs_layout_passes=False, use_tc_tiling_on_sc=False),
    )
    return f(users, movies, user_factors, movie_factors)


# speedup vs baseline: 1.5333x; 1.5333x over previous
import jax
import jax.numpy as jnp
from jax import lax
from jax.experimental import pallas as pl
from jax.experimental.pallas import tpu as pltpu
from jax.experimental.pallas import tpu_sc as plsc

_B = 16384
_D = 20
_P = 128
_NC = 2
_NS = 16
_NW = _NC * _NS
_BPW = _B // _NW
_HALF = _BPW // 2
_CHUNK = 128
_EPS2 = 1e-16


def _rsqrt(p):
    i = plsc.bitcast(p, jnp.int32)
    i = jnp.int32(0x5F3759DF) - (i >> 1)
    y = plsc.bitcast(i, jnp.float32)
    for _ in range(3):
        y = y * (1.5 - 0.5 * p * y * y)
    return y


def _body(users, movies, uf, mf, out, idx_u, idx_m, u_rows, m_rows, out_v, sem):
    wid = lax.axis_index("s") * _NC + lax.axis_index("c")
    base = wid * _BPW
    pltpu.sync_copy(users.at[pl.ds(base, _BPW)], idx_u)
    pltpu.sync_copy(movies.at[pl.ds(base, _BPW)], idx_m)

    lanes = lax.iota(jnp.int32, 16)

    for half in range(2):
        copies = []
        for j in range(_HALF // _CHUNK):
            src_sl = pl.ds(half * _HALF + j * _CHUNK, _CHUNK)
            dst_sl = pl.ds(j * _CHUNK, _CHUNK)
            copies.append(pltpu.async_copy(
                uf.at[idx_u.at[src_sl]], u_rows.at[dst_sl], sem))
            copies.append(pltpu.async_copy(
                mf.at[idx_m.at[src_sl]], m_rows.at[dst_sl], sem))
        for c in copies:
            c.wait()

        def group(g, carry):
            rows = g * 16 + lanes
            dot = jnp.zeros((16,), jnp.float32)
            nu = jnp.zeros((16,), jnp.float32)
            nm = jnp.zeros((16,), jnp.float32)
            for d in range(_D):
                col = jnp.full((16,), d, jnp.int32)
                uv = plsc.load_gather(u_rows, [rows, col])
                mv = plsc.load_gather(m_rows, [rows, col])
                dot = dot + uv * mv
                nu = nu + uv * uv
                nm = nm + mv * mv
            p = jnp.maximum(nu, _EPS2) * jnp.maximum(nm, _EPS2)
            out_v[pl.ds(half * _HALF + g * 16, 16)] = dot * _rsqrt(p) * 2.0 + 3.0
            return carry

        lax.fori_loop(0, _HALF // 16, group, 0)

    pltpu.sync_copy(out_v, out.at[pl.ds(base, _BPW)])


def kernel(users, movies, user_factors, movie_factors):
    ufp = jnp.pad(user_factors, ((0, 0), (0, _P - _D)))
    mfp = jnp.pad(movie_factors, ((0, 0), (0, _P - _D)))
    mesh = plsc.VectorSubcoreMesh(
        core_axis_name="c", subcore_axis_name="s",
        num_cores=_NC, num_subcores=_NS)
    f = pl.kernel(
        _body,
        out_type=jax.ShapeDtypeStruct((_B,), jnp.float32),
        mesh=mesh,
        scratch_types=[
            pltpu.VMEM((_BPW,), jnp.int32),
            pltpu.VMEM((_BPW,), jnp.int32),
            pltpu.VMEM((_HALF, _P), jnp.float32),
            pltpu.VMEM((_HALF, _P), jnp.float32),
            pltpu.VMEM((_BPW,), jnp.float32),
            pltpu.SemaphoreType.DMA,
        ],
        compiler_params=pltpu.CompilerParams(needs_layout_passes=False),
    )
    return f(users, movies, ufp, mfp)


# speedup vs baseline: 2.6776x; 1.7463x over previous
import jax
import jax.numpy as jnp
from jax import lax
from jax.experimental import pallas as pl
from jax.experimental.pallas import tpu as pltpu

_B = 16384
_D = 20
_EPS = 1e-8


def _tc_body(users_s, movies_s, uf, mf, out_v, u_v, m_v, sem):
    def issue(i, carry):
        ru = users_s[i]
        rm = movies_s[i]
        pltpu.make_async_copy(uf.at[pl.ds(ru, 1)], u_v.at[pl.ds(i, 1)], sem).start()
        pltpu.make_async_copy(mf.at[pl.ds(rm, 1)], m_v.at[pl.ds(i, 1)], sem).start()
        return carry

    lax.fori_loop(0, _B, issue, 0, unroll=8)
    pltpu.make_async_copy(uf.at[pl.ds(0, _B)], u_v, sem).wait()
    pltpu.make_async_copy(mf.at[pl.ds(0, _B)], m_v, sem).wait()

    u = u_v[...]
    m = m_v[...]
    dot = jnp.sum(u * m, axis=1)
    nu = jnp.maximum(jnp.sqrt(jnp.sum(u * u, axis=1)), _EPS)
    nm = jnp.maximum(jnp.sqrt(jnp.sum(m * m, axis=1)), _EPS)
    out_v[...] = dot / (nu * nm) * 2.0 + 3.0


def kernel(users, movies, user_factors, movie_factors):
    return pl.pallas_call(
        _tc_body,
        grid=(),
        in_specs=[
            pl.BlockSpec(memory_space=pltpu.SMEM),
            pl.BlockSpec(memory_space=pltpu.SMEM),
            pl.BlockSpec(memory_space=pltpu.HBM),
            pl.BlockSpec(memory_space=pltpu.HBM),
        ],
        out_specs=pl.BlockSpec(memory_space=pltpu.VMEM),
        out_shape=jax.ShapeDtypeStruct((_B,), jnp.float32),
        scratch_shapes=[
            pltpu.VMEM((_B, _D), jnp.float32),
            pltpu.VMEM((_B, _D), jnp.float32),
            pltpu.SemaphoreType.DMA,
        ],
    )(users, movies, user_factors, movie_factors)
